# Initial kernel scaffold; baseline (speedup 1.0000x reference)
import jax
import jax.numpy as jnp
from jax.experimental import pallas as pl


def kernel(x, edge_index, w1, b1, g1, be1, w2, b2, g2, be2, w3, b3, g3, be3):
    raise NotImplementedError("write your pallas kernel here")


# speedup vs baseline: 2.7792x; 2.7792x over previous
import functools
import math

import jax
import jax.numpy as jnp
from jax.experimental import pallas as pl
from jax.experimental.pallas import tpu as pltpu


def _stats(agg, sum_ref, sq_ref):
    sum_ref[...] = jnp.sum(agg, axis=0, keepdims=True)[None]
    sq_ref[...] = jnp.sum(agg * agg, axis=0, keepdims=True)[None]


def _layer1_kernel(x_ref, w_ref, s_ref, agg_ref, sum_ref, sq_ref, *, groups, gin):
    parts = []
    for i in range(groups):
        xg = x_ref[pl.ds(i * gin, gin), :]
        parts.append(jax.lax.dot_general(
            xg, w_ref[...], (((0,), (0,)), ((), ())),
            preferred_element_type=jnp.float32))
    hw = jnp.concatenate(parts, axis=1)
    agg = jnp.dot(s_ref[...], hw, preferred_element_type=jnp.float32)
    agg_ref[...] = agg
    _stats(agg, sum_ref, sq_ref)


def _layer_kernel(h_ref, scale_ref, shift_ref, w_ref, s_ref,
                  agg_ref, sum_ref, sq_ref, *, groups, gin):
    h = jnp.maximum(h_ref[...] * scale_ref[...] + shift_ref[...], 0.0)
    parts = []
    for i in range(groups):
        hg = h[:, i * gin:(i + 1) * gin]
        parts.append(jnp.dot(hg, w_ref[...],
                             preferred_element_type=jnp.float32))
    hw = jnp.concatenate(parts, axis=1)
    agg = jnp.dot(s_ref[...], hw, preferred_element_type=jnp.float32)
    agg_ref[...] = agg
    _stats(agg, sum_ref, sq_ref)


def _bn_relu_kernel(agg_ref, scale_ref, shift_ref, o_ref):
    o_ref[...] = jnp.maximum(agg_ref[...] * scale_ref[...] + shift_ref[...], 0.0)


def _normalized_adjacency(edge_index, num_nodes):
    src, dst = edge_index[0], edge_index[1]
    oh_src = jax.nn.one_hot(src, num_nodes, dtype=jnp.float32)
    oh_dst = jax.nn.one_hot(dst, num_nodes, dtype=jnp.float32)
    a = jax.lax.dot_general(oh_dst, oh_src, (((0,), (0,)), ((), ())))
    a = a + 2.0 * jnp.eye(num_nodes, dtype=jnp.float32)
    deg = jnp.sum(a, axis=1)
    dinv = jnp.where(deg > 0, jax.lax.rsqrt(deg), 0.0)
    return dinv[:, None] * a * dinv[None, :]


def kernel(x, edge_index, w1, b1, g1, be1, w2, b2, g2, be2, w3, b3, g3, be3):
    b, n, c0, l = x.shape
    B = b * n
    c1, c2, c3 = w1.shape[1], w2.shape[1], w3.shape[1]
    chans = (c0, c1, c2, c3)
    n_nodes = B * l
    eps = 1e-5

    g_sz = 256 // c0 if (256 % c0 == 0 and all(c == c0 for c in chans)) else 1

    bt = B
    for cand in (64, 32, 16, 8, 4, 2, 1):
        if B % cand == 0 and cand % g_sz == 0:
            bt = cand
            break
    nt = B // bt
    groups = bt // g_sz

    s = _normalized_adjacency(edge_index, l)

    eye = jnp.eye(g_sz, dtype=jnp.float32)
    wk = (jnp.kron(eye, w1), jnp.kron(eye, w2), jnp.kron(eye, w3))

    x2 = x.reshape(B * c0, l)

    cp = pltpu.CompilerParams(dimension_semantics=("parallel",),
                              vmem_limit_bytes=48 * 1024 * 1024)

    def act_spec(cw):
        return pl.BlockSpec((l, bt * cw), lambda j: (0, j))

    def full_spec(shape):
        nd = len(shape)
        return pl.BlockSpec(tuple(shape), lambda j: (0,) * nd)

    def stats_spec(cw):
        return pl.BlockSpec((1, 1, bt * cw), lambda j: (j, 0, 0))

    def stats_shape(cw):
        return jax.ShapeDtypeStruct((nt, 1, bt * cw), jnp.float32)

    def fold_stats(psum, psq, gamma, beta, cout):
        tot = psum.reshape(-1, cout).sum(axis=0)
        tot2 = psq.reshape(-1, cout).sum(axis=0)
        mean = tot / n_nodes
        var = tot2 / n_nodes - mean * mean
        scale = gamma * jax.lax.rsqrt(var + eps)
        shift = beta - mean * scale
        return (jnp.tile(scale, bt).reshape(1, bt * cout),
                jnp.tile(shift, bt).reshape(1, bt * cout))

    agg1, ps1, pq1 = pl.pallas_call(
        functools.partial(_layer1_kernel, groups=groups, gin=g_sz * c0),
        grid=(nt,),
        in_specs=[pl.BlockSpec((bt * c0, l), lambda j: (j, 0)),
                  full_spec(wk[0].shape), full_spec(s.shape)],
        out_specs=(act_spec(c1), stats_spec(c1), stats_spec(c1)),
        out_shape=(jax.ShapeDtypeStruct((l, B * c1), jnp.float32),
                   stats_shape(c1), stats_shape(c1)),
        compiler_params=cp,
    )(x2, wk[0], s)
    sc1, sh1 = fold_stats(ps1, pq1, g1, be1, c1)

    def run_layer(h, w_blk, cin, cout, scale, shift):
        return pl.pallas_call(
            functools.partial(_layer_kernel, groups=groups, gin=g_sz * cin),
            grid=(nt,),
            in_specs=[act_spec(cin), full_spec(scale.shape),
                      full_spec(shift.shape), full_spec(w_blk.shape),
                      full_spec(s.shape)],
            out_specs=(act_spec(cout), stats_spec(cout), stats_spec(cout)),
            out_shape=(jax.ShapeDtypeStruct((l, B * cout), jnp.float32),
                       stats_shape(cout), stats_shape(cout)),
            compiler_params=cp,
        )(h, scale, shift, w_blk, s)

    agg2, ps2, pq2 = run_layer(agg1, wk[1], c1, c2, sc1, sh1)
    sc2, sh2 = fold_stats(ps2, pq2, g2, be2, c2)
    agg3, ps3, pq3 = run_layer(agg2, wk[2], c2, c3, sc2, sh2)
    sc3, sh3 = fold_stats(ps3, pq3, g3, be3, c3)

    y = pl.pallas_call(
        _bn_relu_kernel,
        grid=(nt,),
        in_specs=[act_spec(c3), full_spec(sc3.shape), full_spec(sh3.shape)],
        out_specs=act_spec(c3),
        out_shape=jax.ShapeDtypeStruct((l, B * c3), jnp.float32),
        compiler_params=cp,
    )(agg3, sc3, sh3)

    return jnp.transpose(y.reshape(l, B, c3), (1, 2, 0))


# speedup vs baseline: 3.1766x; 1.1430x over previous
import functools
import math

import jax
import jax.numpy as jnp
from jax.experimental import pallas as pl
from jax.experimental.pallas import tpu as pltpu


def _stats(agg, sum_ref, sq_ref):
    sum_ref[...] = jnp.sum(agg, axis=0, keepdims=True)[None]
    sq_ref[...] = jnp.sum(agg * agg, axis=0, keepdims=True)[None]


def _layer1_kernel(x_ref, w_ref, s_ref, agg_ref, sum_ref, sq_ref, *, groups, gin):
    parts = []
    for i in range(groups):
        xg = x_ref[pl.ds(i * gin, gin), :]
        parts.append(jax.lax.dot_general(
            xg, w_ref[...], (((0,), (0,)), ((), ())),
            preferred_element_type=jnp.float32))
    hw = jnp.concatenate(parts, axis=1)
    agg = jnp.dot(s_ref[...], hw, preferred_element_type=jnp.float32)
    agg_ref[...] = agg.astype(agg_ref.dtype)
    _stats(agg, sum_ref, sq_ref)


def _layer_kernel(h_ref, scale_ref, shift_ref, w_ref, s_ref,
                  agg_ref, sum_ref, sq_ref, *, groups, gin):
    h = jnp.maximum(h_ref[...].astype(jnp.float32) * scale_ref[...]
                    + shift_ref[...], 0.0)
    parts = []
    for i in range(groups):
        hg = h[:, i * gin:(i + 1) * gin]
        parts.append(jnp.dot(hg, w_ref[...],
                             preferred_element_type=jnp.float32))
    hw = jnp.concatenate(parts, axis=1)
    agg = jnp.dot(s_ref[...], hw, preferred_element_type=jnp.float32)
    agg_ref[...] = agg.astype(agg_ref.dtype)
    _stats(agg, sum_ref, sq_ref)


def _bn_relu_kernel(agg_ref, scale_ref, shift_ref, o_ref):
    o_ref[...] = jnp.maximum(agg_ref[...].astype(jnp.float32) * scale_ref[...]
                             + shift_ref[...], 0.0)


def _normalized_adjacency(edge_index, num_nodes):
    src, dst = edge_index[0], edge_index[1]
    oh_src = jax.nn.one_hot(src, num_nodes, dtype=jnp.float32)
    oh_dst = jax.nn.one_hot(dst, num_nodes, dtype=jnp.float32)
    a = jax.lax.dot_general(oh_dst, oh_src, (((0,), (0,)), ((), ())))
    a = a + 2.0 * jnp.eye(num_nodes, dtype=jnp.float32)
    deg = jnp.sum(a, axis=1)
    dinv = jnp.where(deg > 0, jax.lax.rsqrt(deg), 0.0)
    return dinv[:, None] * a * dinv[None, :]


def kernel(x, edge_index, w1, b1, g1, be1, w2, b2, g2, be2, w3, b3, g3, be3):
    b, n, c0, l = x.shape
    B = b * n
    c1, c2, c3 = w1.shape[1], w2.shape[1], w3.shape[1]
    chans = (c0, c1, c2, c3)
    n_nodes = B * l
    eps = 1e-5

    g_sz = 256 // c0 if (256 % c0 == 0 and all(c == c0 for c in chans)) else 1

    bt = B
    for cand in (64, 32, 16, 8, 4, 2, 1):
        if B % cand == 0 and cand % g_sz == 0:
            bt = cand
            break
    nt = B // bt
    groups = bt // g_sz

    s = _normalized_adjacency(edge_index, l)

    eye = jnp.eye(g_sz, dtype=jnp.float32)
    wk = (jnp.kron(eye, w1), jnp.kron(eye, w2), jnp.kron(eye, w3))

    x2 = x.reshape(B * c0, l)

    cp = pltpu.CompilerParams(dimension_semantics=("parallel",),
                              vmem_limit_bytes=48 * 1024 * 1024)

    def act_spec(cw):
        return pl.BlockSpec((l, bt * cw), lambda j: (0, j))

    def full_spec(shape):
        nd = len(shape)
        return pl.BlockSpec(tuple(shape), lambda j: (0,) * nd)

    def stats_spec(cw):
        return pl.BlockSpec((1, 1, bt * cw), lambda j: (j, 0, 0))

    def stats_shape(cw):
        return jax.ShapeDtypeStruct((nt, 1, bt * cw), jnp.float32)

    def fold_stats(psum, psq, gamma, beta, cout):
        tot = psum.reshape(-1, cout).sum(axis=0)
        tot2 = psq.reshape(-1, cout).sum(axis=0)
        mean = tot / n_nodes
        var = tot2 / n_nodes - mean * mean
        scale = gamma * jax.lax.rsqrt(var + eps)
        shift = beta - mean * scale
        return (jnp.tile(scale, bt).reshape(1, bt * cout),
                jnp.tile(shift, bt).reshape(1, bt * cout))

    act_dtype = jnp.bfloat16

    agg1, ps1, pq1 = pl.pallas_call(
        functools.partial(_layer1_kernel, groups=groups, gin=g_sz * c0),
        grid=(nt,),
        in_specs=[pl.BlockSpec((bt * c0, l), lambda j: (j, 0)),
                  full_spec(wk[0].shape), full_spec(s.shape)],
        out_specs=(act_spec(c1), stats_spec(c1), stats_spec(c1)),
        out_shape=(jax.ShapeDtypeStruct((l, B * c1), act_dtype),
                   stats_shape(c1), stats_shape(c1)),
        compiler_params=cp,
    )(x2, wk[0], s)
    sc1, sh1 = fold_stats(ps1, pq1, g1, be1, c1)

    def run_layer(h, w_blk, cin, cout, scale, shift):
        return pl.pallas_call(
            functools.partial(_layer_kernel, groups=groups, gin=g_sz * cin),
            grid=(nt,),
            in_specs=[act_spec(cin), full_spec(scale.shape),
                      full_spec(shift.shape), full_spec(w_blk.shape),
                      full_spec(s.shape)],
            out_specs=(act_spec(cout), stats_spec(cout), stats_spec(cout)),
            out_shape=(jax.ShapeDtypeStruct((l, B * cout), act_dtype),
                       stats_shape(cout), stats_shape(cout)),
            compiler_params=cp,
        )(h, scale, shift, w_blk, s)

    agg2, ps2, pq2 = run_layer(agg1, wk[1], c1, c2, sc1, sh1)
    sc2, sh2 = fold_stats(ps2, pq2, g2, be2, c2)
    agg3, ps3, pq3 = run_layer(agg2, wk[2], c2, c3, sc2, sh2)
    sc3, sh3 = fold_stats(ps3, pq3, g3, be3, c3)

    y = pl.pallas_call(
        _bn_relu_kernel,
        grid=(nt,),
        in_specs=[act_spec(c3), full_spec(sc3.shape), full_spec(sh3.shape)],
        out_specs=act_spec(c3),
        out_shape=jax.ShapeDtypeStruct((l, B * c3), jnp.float32),
        compiler_params=cp,
    )(agg3, sc3, sh3)

    return jnp.transpose(y.reshape(l, B, c3), (1, 2, 0))


# speedup vs baseline: 4.0301x; 1.2687x over previous
import functools
import math

import jax
import jax.numpy as jnp
from jax.experimental import pallas as pl
from jax.experimental.pallas import tpu as pltpu


def _stats(agg, sum_ref, sq_ref):
    sum_ref[...] = jnp.sum(agg, axis=0, keepdims=True)[None]
    sq_ref[...] = jnp.sum(agg * agg, axis=0, keepdims=True)[None]


def _layer1_kernel(x_ref, w_ref, s_ref, agg_ref, sum_ref, sq_ref, *, groups, gin):
    parts = []
    for i in range(groups):
        xg = x_ref[pl.ds(i * gin, gin), :]
        parts.append(jax.lax.dot_general(
            xg, w_ref[...], (((0,), (0,)), ((), ())),
            preferred_element_type=jnp.float32))
    hw = jnp.concatenate(parts, axis=1)
    agg = jnp.dot(s_ref[...], hw, preferred_element_type=jnp.float32)
    agg_ref[...] = agg.astype(agg_ref.dtype)
    _stats(agg, sum_ref, sq_ref)


def _layer_kernel(h_ref, scale_ref, shift_ref, w_ref, s_ref,
                  agg_ref, sum_ref, sq_ref, *, groups, gin):
    h = jnp.maximum(h_ref[...].astype(jnp.float32) * scale_ref[...]
                    + shift_ref[...], 0.0)
    parts = []
    for i in range(groups):
        hg = h[:, i * gin:(i + 1) * gin]
        parts.append(jnp.dot(hg, w_ref[...],
                             preferred_element_type=jnp.float32))
    hw = jnp.concatenate(parts, axis=1)
    agg = jnp.dot(s_ref[...], hw, preferred_element_type=jnp.float32)
    agg_ref[...] = agg.astype(agg_ref.dtype)
    _stats(agg, sum_ref, sq_ref)


def _bn_relu_t_kernel(agg_ref, scale_ref, shift_ref, eye_ref, o_ref, *,
                      groups, gout, bt, cout):
    y = jnp.maximum(agg_ref[...].astype(jnp.float32) * scale_ref[...]
                    + shift_ref[...], 0.0)
    parts = []
    for i in range(groups):
        yg = y[:, i * gout:(i + 1) * gout]
        parts.append(jax.lax.dot_general(
            yg, eye_ref[...], (((0,), (0,)), ((), ())),
            preferred_element_type=jnp.float32))
    o_ref[...] = jnp.concatenate(parts, axis=0).reshape(bt, cout, -1)


def _normalized_adjacency(edge_index, num_nodes):
    src, dst = edge_index[0], edge_index[1]
    oh_src = jax.nn.one_hot(src, num_nodes, dtype=jnp.float32)
    oh_dst = jax.nn.one_hot(dst, num_nodes, dtype=jnp.float32)
    a = jax.lax.dot_general(oh_dst, oh_src, (((0,), (0,)), ((), ())))
    a = a + 2.0 * jnp.eye(num_nodes, dtype=jnp.float32)
    deg = jnp.sum(a, axis=1)
    dinv = jnp.where(deg > 0, jax.lax.rsqrt(deg), 0.0)
    return dinv[:, None] * a * dinv[None, :]


def kernel(x, edge_index, w1, b1, g1, be1, w2, b2, g2, be2, w3, b3, g3, be3):
    b, n, c0, l = x.shape
    B = b * n
    c1, c2, c3 = w1.shape[1], w2.shape[1], w3.shape[1]
    chans = (c0, c1, c2, c3)
    n_nodes = B * l
    eps = 1e-5

    g_sz = 256 // c0 if (256 % c0 == 0 and all(c == c0 for c in chans)) else 1

    bt = B
    for cand in (64, 32, 16, 8, 4, 2, 1):
        if B % cand == 0 and cand % g_sz == 0:
            bt = cand
            break
    nt = B // bt
    groups = bt // g_sz

    s = _normalized_adjacency(edge_index, l)

    eye = jnp.eye(g_sz, dtype=jnp.float32)
    wk = (jnp.kron(eye, w1), jnp.kron(eye, w2), jnp.kron(eye, w3))

    x2 = x.reshape(B * c0, l)

    cp = pltpu.CompilerParams(dimension_semantics=("parallel",),
                              vmem_limit_bytes=48 * 1024 * 1024)

    def act_spec(cw):
        return pl.BlockSpec((l, bt * cw), lambda j: (0, j))

    def full_spec(shape):
        nd = len(shape)
        return pl.BlockSpec(tuple(shape), lambda j: (0,) * nd)

    def stats_spec(cw):
        return pl.BlockSpec((1, 1, bt * cw), lambda j: (j, 0, 0))

    def stats_shape(cw):
        return jax.ShapeDtypeStruct((nt, 1, bt * cw), jnp.float32)

    def fold_stats(psum, psq, gamma, beta, cout):
        tot = psum.reshape(-1, cout).sum(axis=0)
        tot2 = psq.reshape(-1, cout).sum(axis=0)
        mean = tot / n_nodes
        var = tot2 / n_nodes - mean * mean
        scale = gamma * jax.lax.rsqrt(var + eps)
        shift = beta - mean * scale
        return (jnp.tile(scale, bt).reshape(1, bt * cout),
                jnp.tile(shift, bt).reshape(1, bt * cout))

    act_dtype = jnp.bfloat16

    agg1, ps1, pq1 = pl.pallas_call(
        functools.partial(_layer1_kernel, groups=groups, gin=g_sz * c0),
        grid=(nt,),
        in_specs=[pl.BlockSpec((bt * c0, l), lambda j: (j, 0)),
                  full_spec(wk[0].shape), full_spec(s.shape)],
        out_specs=(act_spec(c1), stats_spec(c1), stats_spec(c1)),
        out_shape=(jax.ShapeDtypeStruct((l, B * c1), act_dtype),
                   stats_shape(c1), stats_shape(c1)),
        compiler_params=cp,
    )(x2, wk[0], s)
    sc1, sh1 = fold_stats(ps1, pq1, g1, be1, c1)

    def run_layer(h, w_blk, cin, cout, scale, shift):
        return pl.pallas_call(
            functools.partial(_layer_kernel, groups=groups, gin=g_sz * cin),
            grid=(nt,),
            in_specs=[act_spec(cin), full_spec(scale.shape),
                      full_spec(shift.shape), full_spec(w_blk.shape),
                      full_spec(s.shape)],
            out_specs=(act_spec(cout), stats_spec(cout), stats_spec(cout)),
            out_shape=(jax.ShapeDtypeStruct((l, B * cout), act_dtype),
                       stats_shape(cout), stats_shape(cout)),
            compiler_params=cp,
        )(h, scale, shift, w_blk, s)

    agg2, ps2, pq2 = run_layer(agg1, wk[1], c1, c2, sc1, sh1)
    sc2, sh2 = fold_stats(ps2, pq2, g2, be2, c2)
    agg3, ps3, pq3 = run_layer(agg2, wk[2], c2, c3, sc2, sh2)
    sc3, sh3 = fold_stats(ps3, pq3, g3, be3, c3)

    eye_l = jnp.eye(l, dtype=jnp.float32)
    y = pl.pallas_call(
        functools.partial(_bn_relu_t_kernel, groups=groups, gout=g_sz * c3,
                          bt=bt, cout=c3),
        grid=(nt,),
        in_specs=[act_spec(c3), full_spec(sc3.shape), full_spec(sh3.shape),
                  full_spec(eye_l.shape)],
        out_specs=pl.BlockSpec((bt, c3, l), lambda j: (j, 0, 0)),
        out_shape=jax.ShapeDtypeStruct((B, c3, l), jnp.float32),
        compiler_params=cp,
    )(agg3, sc3, sh3, eye_l)

    return y
